# baseline (speedup 1.0000x reference)
import functools

import jax
import jax.numpy as jnp
from jax import lax
from jax.experimental import pallas as pl
from jax.experimental.pallas import tpu as pltpu
from jax.experimental.pallas import tpu_sc as plsc

D = 64
LANES = 16
CHUNK = 512


def _sc_kernel(n_rows, nw, gid_hbm, gval_hbm, table_hbm, w_hbm, b_hbm, out_hbm,
               idx_v, gval_v, rows_v, wb_v, sem):
    nc = 2
    wid = lax.axis_index("s") * nc + lax.axis_index("c")
    per_w = n_rows // nw
    n_chunks = per_w // CHUNK

    pltpu.sync_copy(w_hbm, wb_v.at[0])
    pltpu.sync_copy(b_hbm, wb_v.at[1])
    wvec = [wb_v[0, pl.ds(t * LANES, LANES)] for t in range(D // LANES)]
    bvec = [wb_v[1, pl.ds(t * LANES, LANES)] for t in range(D // LANES)]

    def chunk_body(c, _):
        base = wid * per_w + c * CHUNK
        pltpu.sync_copy(gid_hbm.at[pl.ds(base, CHUNK)], idx_v)
        pltpu.sync_copy(gval_hbm.at[pl.ds(base, CHUNK)], gval_v)
        pltpu.async_copy(table_hbm.at[idx_v], rows_v, sem).wait()

        def grp_body(j, _):
            gvec = gval_v[pl.ds(j * LANES, LANES)]
            for k in range(LANES):
                i = j * LANES + k
                s = gvec[k]
                for t in range(D // LANES):
                    sl = pl.ds(t * LANES, LANES)
                    rows_v[i, sl] = rows_v[i, sl] + s * wvec[t] + bvec[t]
            return 0

        lax.fori_loop(0, CHUNK // LANES, grp_body, 0)
        pltpu.sync_copy(rows_v, out_hbm.at[pl.ds(base, CHUNK)])
        return 0

    lax.fori_loop(0, n_chunks, chunk_body, 0)


def kernel(gene_id, gene_val, emb_table, w_val, b_val):
    b, l = gene_id.shape
    n = b * l
    gid = jnp.reshape(gene_id, (n,)).astype(jnp.int32)
    gval = jnp.reshape(gene_val, (n,))

    info = plsc.get_sparse_core_info()
    nw = info.num_cores * info.num_subcores

    mesh = plsc.VectorSubcoreMesh(core_axis_name="c", subcore_axis_name="s")
    run = pl.kernel(
        functools.partial(_sc_kernel, n, nw),
        mesh=mesh,
        compiler_params=pltpu.CompilerParams(use_tc_tiling_on_sc=False),
        out_type=jax.ShapeDtypeStruct((n, D), jnp.float32),
        scratch_types=[
            pltpu.VMEM((CHUNK,), jnp.int32),
            pltpu.VMEM((CHUNK,), jnp.float32),
            pltpu.VMEM((CHUNK, D), jnp.float32),
            pltpu.VMEM((2, D), jnp.float32),
            pltpu.SemaphoreType.DMA,
        ],
    )
    out = run(gid, gval, emb_table, w_val, b_val)
    return jnp.reshape(out, (b, l, D))


# speedup vs baseline: 1.0985x; 1.0985x over previous
import functools

import jax
import jax.numpy as jnp
from jax import lax
from jax.experimental import pallas as pl
from jax.experimental.pallas import tpu as pltpu
from jax.experimental.pallas import tpu_sc as plsc

D = 64
LANES = 16
CHUNK = 400
NBUF = 2


def _sc_kernel(n_rows, nw, gid_hbm, gval_hbm, table_hbm, w_hbm, b_hbm, out_hbm,
               idx_v, gval_v, gbuf, obuf, wb_v, gsem0, gsem1, osem0, osem1):
    nc = 2
    wid = lax.axis_index("s") * nc + lax.axis_index("c")
    per_w = n_rows // nw
    n_chunks = per_w // CHUNK
    nsuper = n_chunks // NBUF
    base0 = wid * per_w
    gsem = [gsem0, gsem1]
    osem = [osem0, osem1]

    pltpu.sync_copy(w_hbm, wb_v.at[0])
    pltpu.sync_copy(b_hbm, wb_v.at[1])
    wvec = [wb_v[0, pl.ds(t * LANES, LANES)] for t in range(D // LANES)]
    bvec = [wb_v[1, pl.ds(t * LANES, LANES)] for t in range(D // LANES)]

    def start_gather(b, c):
        base = base0 + c * CHUNK
        pltpu.sync_copy(gid_hbm.at[pl.ds(base, CHUNK)], idx_v.at[b])
        pltpu.sync_copy(gval_hbm.at[pl.ds(base, CHUNK)], gval_v.at[b])
        pltpu.async_copy(table_hbm.at[idx_v.at[b]], gbuf.at[b], gsem[b])

    def wait_gather(b):
        pltpu.make_async_copy(table_hbm.at[idx_v.at[b]], gbuf.at[b],
                              gsem[b]).wait()

    def start_wb(b, c):
        base = base0 + c * CHUNK
        pltpu.async_copy(obuf.at[b], out_hbm.at[pl.ds(base, CHUNK)], osem[b])

    def wait_wb(b, c):
        base = base0 + c * CHUNK
        pltpu.make_async_copy(obuf.at[b], out_hbm.at[pl.ds(base, CHUNK)],
                              osem[b]).wait()

    def compute(b):
        def grp_body(j, _):
            gvec = gval_v[b, pl.ds(j * LANES, LANES)]
            for k in range(LANES):
                i = j * LANES + k
                s = gvec[k]
                for t in range(D // LANES):
                    sl = pl.ds(t * LANES, LANES)
                    obuf[b, i, sl] = gbuf[b, i, sl] + (s * wvec[t] + bvec[t])
            return 0

        lax.fori_loop(0, CHUNK // LANES, grp_body, 0)

    for b in range(NBUF):
        start_gather(b, b)

    def super_body(s, _):
        for b in range(NBUF):
            c = s * NBUF + b
            wait_gather(b)

            @pl.when(s >= 1)
            def _():
                wait_wb(b, c - NBUF)

            compute(b)
            start_wb(b, c)

            @pl.when(s < nsuper - 1)
            def _():
                start_gather(b, c + NBUF)

        return 0

    lax.fori_loop(0, nsuper, super_body, 0)

    for b in range(NBUF):
        wait_wb(b, n_chunks - NBUF + b)


def kernel(gene_id, gene_val, emb_table, w_val, b_val):
    b, l = gene_id.shape
    n = b * l
    gid = jnp.reshape(gene_id, (n,)).astype(jnp.int32)
    gval = jnp.reshape(gene_val, (n,))

    info = plsc.get_sparse_core_info()
    nw = info.num_cores * info.num_subcores

    mesh = plsc.VectorSubcoreMesh(core_axis_name="c", subcore_axis_name="s")
    run = pl.kernel(
        functools.partial(_sc_kernel, n, nw),
        mesh=mesh,
        compiler_params=pltpu.CompilerParams(use_tc_tiling_on_sc=False),
        out_type=jax.ShapeDtypeStruct((n, D), jnp.float32),
        scratch_types=[
            pltpu.VMEM((NBUF, CHUNK), jnp.int32),
            pltpu.VMEM((NBUF, CHUNK), jnp.float32),
            pltpu.VMEM((NBUF, CHUNK, D), jnp.float32),
            pltpu.VMEM((NBUF, CHUNK, D), jnp.float32),
            pltpu.VMEM((2, D), jnp.float32),
            pltpu.SemaphoreType.DMA,
            pltpu.SemaphoreType.DMA,
            pltpu.SemaphoreType.DMA,
            pltpu.SemaphoreType.DMA,
        ],
    )
    out = run(gid, gval, emb_table, w_val, b_val)
    return jnp.reshape(out, (b, l, D))
